# baseline (speedup 1.0000x reference)
import jax
import jax.numpy as jnp
from jax.experimental import pallas as pl
from jax.experimental.pallas import tpu as pltpu

VOCAB = 96
VPAD = 128
ECHAR = 50
EWORD = 64
KSIZE = 5


def _fused_kernel(nb, mw):
    npos = mw - KSIZE + 1
    nquads = npos // 4

    def body(idx_ref, emb_ref, wflat_ref, wpg_ref, cb1_ref, bpg_ref,
             out_ref, tcat_ref, tquad_ref):
        @pl.when(pl.program_id(0) == 0)
        def _build_tables():
            emb = emb_ref[...]
            z32 = jnp.zeros((VPAD - VOCAB, EWORD), jnp.float32)
            pieces = []
            for k in range(KSIZE):
                tk = emb @ wflat_ref[k * ECHAR:(k + 1) * ECHAR, :]
                pieces.append(tk)
                pieces.append(z32)
            tcat = jnp.concatenate(pieces, axis=0)
            tcat_ref[...] = tcat.astype(jnp.bfloat16)
            z128 = jnp.zeros((VPAD, EWORD), jnp.float32)
            cols = []
            for q in range(4):
                col = jnp.concatenate([z128] * q + [tcat] + [z128] * (3 - q),
                                      axis=0)
                cols.append(col)
            tquad_ref[...] = jnp.concatenate(cols, axis=1).astype(jnp.bfloat16)

        idx = idx_ref[...]
        iota = jax.lax.broadcasted_iota(jnp.int32, (nb, VPAD), 1)
        iotab = iota.astype(jnp.bfloat16)
        oh = jnp.concatenate(
            [(idx[:, j][:, None] == iotab).astype(jnp.bfloat16)
             for j in range(mw)], axis=1)

        tquad = tquad_ref[...]
        m256 = None
        for q in range(nquads):
            base = 4 * q * VPAD
            a = jax.lax.dot_general(
                oh[:, base:base + 8 * VPAD], tquad,
                (((1,), (0,)), ((), ())),
                preferred_element_type=jnp.float32)
            m256 = a if m256 is None else jnp.maximum(m256, a)
        m = jnp.maximum(
            jnp.maximum(m256[:, :EWORD], m256[:, EWORD:2 * EWORD]),
            jnp.maximum(m256[:, 2 * EWORD:3 * EWORD], m256[:, 3 * EWORD:]))
        for t in range(4 * nquads, npos):
            base = t * VPAD
            a = jax.lax.dot_general(
                oh[:, base:base + KSIZE * VPAD], tcat_ref[...],
                (((1,), (0,)), ((), ())),
                preferred_element_type=jnp.float32)
            m = jnp.maximum(m, a)
        m = jnp.maximum(m + cb1_ref[...], 0.0)

        h = jax.lax.dot_general(
            m.astype(jnp.bfloat16), wpg_ref[...], (((1,), (0,)), ((), ())),
            preferred_element_type=jnp.float32) + bpg_ref[...]
        proj = jnp.maximum(h[:, :EWORD], 0.0)
        gate = jax.nn.sigmoid(h[:, EWORD:])
        out_ref[...] = gate * proj + (1.0 - gate) * m

    return body


def kernel(input, emb_table, conv_w, conv_b, W_proj, b_proj, W_gate, b_gate):
    sl, bs, mw = input.shape
    n = sl * bs
    idx = input.reshape(n, mw).astype(jnp.bfloat16)

    wflat = conv_w.transpose(2, 1, 0).reshape(KSIZE * ECHAR, EWORD)
    wpg = jnp.concatenate([W_proj.T, W_gate.T], axis=1).astype(jnp.bfloat16)
    cb1 = conv_b[None, :]
    bpg = jnp.concatenate([b_proj, b_gate])[None, :]

    nb = 2048 if n % 2048 == 0 else n
    grid = (n // nb,)

    out = pl.pallas_call(
        _fused_kernel(nb, mw),
        grid=grid,
        in_specs=[
            pl.BlockSpec((nb, mw), lambda i: (i, 0)),
            pl.BlockSpec((VOCAB, ECHAR), lambda i: (0, 0)),
            pl.BlockSpec((KSIZE * ECHAR, EWORD), lambda i: (0, 0)),
            pl.BlockSpec((EWORD, 2 * EWORD), lambda i: (0, 0)),
            pl.BlockSpec((1, EWORD), lambda i: (0, 0)),
            pl.BlockSpec((1, 2 * EWORD), lambda i: (0, 0)),
        ],
        out_specs=pl.BlockSpec((nb, EWORD), lambda i: (i, 0)),
        out_shape=jax.ShapeDtypeStruct((n, EWORD), jnp.float32),
        scratch_shapes=[
            pltpu.VMEM((KSIZE * VPAD, EWORD), jnp.bfloat16),
            pltpu.VMEM((8 * VPAD, 4 * EWORD), jnp.bfloat16),
        ],
    )(idx, emb_table, wflat, wpg, cb1, bpg)
    return out.reshape(sl, bs, EWORD)


# speedup vs baseline: 1.0099x; 1.0099x over previous
import jax
import jax.numpy as jnp
from jax.experimental import pallas as pl
from jax.experimental.pallas import tpu as pltpu

VOCAB = 96
VPAD = 128
ECHAR = 50
EWORD = 64
KSIZE = 5


def _fused_kernel(nb, mw):
    npos = mw - KSIZE + 1
    nquads = npos // 4

    def body(idx_ref, emb_ref, wflat_ref, wpg_ref, cb1_ref, bpg_ref,
             out_ref, tcat_ref, tquad_ref):
        @pl.when(pl.program_id(0) == 0)
        def _build_tables():
            emb = emb_ref[...]
            z32 = jnp.zeros((VPAD - VOCAB, EWORD), jnp.float32)
            pieces = []
            for k in range(KSIZE):
                tk = emb @ wflat_ref[k * ECHAR:(k + 1) * ECHAR, :]
                pieces.append(tk)
                pieces.append(z32)
            tcat = jnp.concatenate(pieces, axis=0)
            tcat_ref[...] = tcat.astype(jnp.bfloat16)
            z128 = jnp.zeros((VPAD, EWORD), jnp.float32)
            cols = []
            for q in range(4):
                col = jnp.concatenate([z128] * q + [tcat] + [z128] * (3 - q),
                                      axis=0)
                cols.append(col)
            tquad_ref[...] = jnp.concatenate(cols, axis=1).astype(jnp.bfloat16)

        idx = idx_ref[...]
        iota = jax.lax.broadcasted_iota(jnp.int32, (nb, VPAD), 1)
        iotab = iota.astype(jnp.bfloat16)
        one = jnp.ones((nb, VPAD), jnp.bfloat16)
        zero = jnp.zeros((nb, VPAD), jnp.bfloat16)
        oh = jnp.concatenate(
            [jnp.where(idx[:, j][:, None] == iotab, one, zero)
             for j in range(mw)], axis=1)

        tquad = tquad_ref[...]
        m256 = None
        for q in range(nquads):
            base = 4 * q * VPAD
            a = jax.lax.dot_general(
                oh[:, base:base + 8 * VPAD], tquad,
                (((1,), (0,)), ((), ())),
                preferred_element_type=jnp.float32)
            m256 = a if m256 is None else jnp.maximum(m256, a)
        m = jnp.maximum(
            jnp.maximum(m256[:, :EWORD], m256[:, EWORD:2 * EWORD]),
            jnp.maximum(m256[:, 2 * EWORD:3 * EWORD], m256[:, 3 * EWORD:]))
        for t in range(4 * nquads, npos):
            base = t * VPAD
            a = jax.lax.dot_general(
                oh[:, base:base + KSIZE * VPAD], tcat_ref[...],
                (((1,), (0,)), ((), ())),
                preferred_element_type=jnp.float32)
            m = jnp.maximum(m, a)
        m = jnp.maximum(m + cb1_ref[...], 0.0)

        h = jax.lax.dot_general(
            m.astype(jnp.bfloat16), wpg_ref[...], (((1,), (0,)), ((), ())),
            preferred_element_type=jnp.float32) + bpg_ref[...]
        proj = jnp.maximum(h[:, :EWORD], 0.0)
        gate = jax.nn.sigmoid(h[:, EWORD:])
        out_ref[...] = gate * proj + (1.0 - gate) * m

    return body


def kernel(input, emb_table, conv_w, conv_b, W_proj, b_proj, W_gate, b_gate):
    sl, bs, mw = input.shape
    n = sl * bs
    idx = input.reshape(n, mw).astype(jnp.bfloat16)

    wflat = conv_w.transpose(2, 1, 0).reshape(KSIZE * ECHAR, EWORD)
    wpg = jnp.concatenate([W_proj.T, W_gate.T], axis=1).astype(jnp.bfloat16)
    cb1 = conv_b[None, :]
    bpg = jnp.concatenate([b_proj, b_gate])[None, :]

    nb = 2048 if n % 2048 == 0 else n
    grid = (n // nb,)

    out = pl.pallas_call(
        _fused_kernel(nb, mw),
        grid=grid,
        in_specs=[
            pl.BlockSpec((nb, mw), lambda i: (i, 0)),
            pl.BlockSpec((VOCAB, ECHAR), lambda i: (0, 0)),
            pl.BlockSpec((KSIZE * ECHAR, EWORD), lambda i: (0, 0)),
            pl.BlockSpec((EWORD, 2 * EWORD), lambda i: (0, 0)),
            pl.BlockSpec((1, EWORD), lambda i: (0, 0)),
            pl.BlockSpec((1, 2 * EWORD), lambda i: (0, 0)),
        ],
        out_specs=pl.BlockSpec((nb, EWORD), lambda i: (i, 0)),
        out_shape=jax.ShapeDtypeStruct((n, EWORD), jnp.float32),
        scratch_shapes=[
            pltpu.VMEM((KSIZE * VPAD, EWORD), jnp.bfloat16),
            pltpu.VMEM((8 * VPAD, 4 * EWORD), jnp.bfloat16),
        ],
    )(idx, emb_table, wflat, wpg, cb1, bpg)
    return out.reshape(sl, bs, EWORD)
